# Initial kernel scaffold; baseline (speedup 1.0000x reference)
import jax
import jax.numpy as jnp
from jax.experimental import pallas as pl


def kernel(z, codebook):
    raise NotImplementedError("write your pallas kernel here")


# speedup vs baseline: 1.6641x; 1.6641x over previous
import functools

import jax
import jax.numpy as jnp
from jax import lax
from jax.experimental import pallas as pl
from jax.experimental.pallas import tpu as pltpu
from jax.experimental.pallas import tpu_sc as plsc

_TT = 1024


def _dist_body(c_ref, zt_ref, zsq_ref, csq_ref, idx_ref):
    c = c_ref[...]
    zt = zt_ref[...]
    k = c.shape[0]
    zsq = zsq_ref[0:1, :]
    csq = csq_ref[:, 0:1]
    zc = lax.dot_general(c.astype(jnp.bfloat16), zt.astype(jnp.bfloat16),
                         (((1,), (0,)), ((), ())),
                         preferred_element_type=jnp.float32)
    d2 = (zsq + csq) - 2.0 * zc
    dist = jnp.sqrt(jnp.maximum(d2, 0.0))
    m = jnp.min(dist, axis=0, keepdims=True)
    iota = lax.broadcasted_iota(jnp.int32, dist.shape, 0)
    cand = jnp.where(dist == m, iota, k)
    idx_ref[...] = jnp.min(cand, axis=0).astype(jnp.int32)


def _indices_tc(c, zt, zsq8, csq8):
    k, d = c.shape
    n = zt.shape[1]
    return pl.pallas_call(
        _dist_body,
        grid=(n // _TT,),
        in_specs=[
            pl.BlockSpec((k, d), lambda i: (0, 0)),
            pl.BlockSpec((d, _TT), lambda i: (0, i)),
            pl.BlockSpec((8, _TT), lambda i: (0, i)),
            pl.BlockSpec((k, 8), lambda i: (0, 0)),
        ],
        out_specs=pl.BlockSpec((_TT,), lambda i: (i,)),
        out_shape=jax.ShapeDtypeStruct((n,), jnp.int32),
    )(c, zt, zsq8, csq8)


@functools.lru_cache(maxsize=None)
def _make_gather(n, d):
    info = plsc.get_sparse_core_info()
    nc, ns = info.num_cores, info.num_subcores
    nw = nc * ns
    bpw = n // nw
    ch = 128
    nch = bpw // ch
    mesh = plsc.VectorSubcoreMesh(core_axis_name="c", subcore_axis_name="s")

    @functools.partial(
        pl.kernel, mesh=mesh,
        out_type=jax.ShapeDtypeStruct((n, d), jnp.float32),
        compiler_params=pltpu.CompilerParams(use_tc_tiling_on_sc=False),
        scratch_types=[
            pltpu.VMEM((nch, ch), jnp.int32),
            pltpu.VMEM((bpw, d), jnp.float32),
            pltpu.SemaphoreType.DMA,
        ],
    )
    def gather(cb_hbm, idx_hbm, out_hbm, idx_v, rows_v, sem):
        wid = lax.axis_index("s") * nc + lax.axis_index("c")
        base = wid * bpw
        pltpu.sync_copy(idx_hbm.at[wid], idx_v)
        cps = [pltpu.async_copy(cb_hbm.at[idx_v.at[j]],
                                rows_v.at[pl.ds(j * ch, ch)], sem)
               for j in range(nch)]
        for cp in cps:
            cp.wait()
        pltpu.sync_copy(rows_v, out_hbm.at[pl.ds(base, bpw)])

    return gather, nw, nch, ch


def kernel(z, codebook):
    n, d = z.shape
    k = codebook.shape[0]
    zsq8 = jnp.broadcast_to(jnp.sum(z * z, axis=-1)[None, :], (8, n))
    csq8 = jnp.broadcast_to(jnp.sum(codebook * codebook, axis=-1)[:, None], (k, 8))
    indices = _indices_tc(codebook, z.T, zsq8, csq8)
    gather, nw, nch, ch = _make_gather(n, d)
    z_q = gather(codebook, indices.reshape(nw, nch, ch))
    return (z_q, indices)
